# baseline (speedup 1.0000x reference)
import jax
import jax.numpy as jnp
from jax.experimental import pallas as pl
from jax.experimental.pallas import tpu as pltpu

_R = 2


def _hard_attention_kernel(feat_ref, hid_ref, wft_ref, bf_ref, wh_ref, bh_ref,
                           wblk_ref, alpha_ref, z_ref):
    b = pl.program_id(0)
    n = feat_ref.shape[1]
    dn = (((1,), (1,)), ((), ()))
    ones_row = jnp.ones((1, n), jnp.float32)

    us = []
    for r in range(_R):
        feat = feat_ref[r]
        hrow = hid_ref[pl.ds(b * _R + r, 1), 0, :]

        ft = jax.lax.dot_general(wft_ref[...], feat, dn,
                                 preferred_element_type=jnp.float32)
        hb_row = (jnp.dot(hrow, wh_ref[...],
                          preferred_element_type=jnp.float32)
                  + bh_ref[...] + bf_ref[...])
        hb_bc = jax.lax.dot_general(hb_row, ones_row, (((0,), (0,)), ((), ())),
                                    preferred_element_type=jnp.float32,
                                    precision=jax.lax.Precision.HIGHEST)
        us.append(jnp.tanh(ft + hb_bc))

    ucat = jnp.concatenate(us, axis=0)
    lo = jnp.dot(wblk_ref[...], ucat,
                 preferred_element_type=jnp.float32)

    m = jnp.max(lo, axis=1, keepdims=True)
    e = jnp.exp(lo - m)
    s = jnp.sum(e, axis=1, keepdims=True)
    alpha_ref[:, 0, :] = e * (1.0 / s)

    iota = jax.lax.broadcasted_iota(jnp.int32, (_R, n), 1)
    bidx = jnp.min(jnp.where(lo == m, iota, n), axis=1)
    for r in range(_R):
        z_ref[r] = feat_ref[r, pl.ds(bidx[r], 1), :]


def kernel(features, hidden, Wf, bf, Wh, bh, Ws, bs):
    B, N, D = features.shape
    H = hidden.shape[1]
    U = Wf.shape[1]

    hidden3 = hidden.reshape(B, 1, H)
    wft = Wf.T
    bf2 = bf.reshape(1, U)
    bh2 = bh.reshape(1, U)
    wblk = jnp.kron(jnp.eye(_R, dtype=jnp.float32), Ws.reshape(1, U))

    alpha2, z3 = pl.pallas_call(
        _hard_attention_kernel,
        grid=(B // _R,),
        in_specs=[
            pl.BlockSpec((_R, N, D), lambda b: (b, 0, 0)),
            pl.BlockSpec((B, 1, H), lambda b: (0, 0, 0)),
            pl.BlockSpec((U, D), lambda b: (0, 0)),
            pl.BlockSpec((1, U), lambda b: (0, 0)),
            pl.BlockSpec((H, U), lambda b: (0, 0)),
            pl.BlockSpec((1, U), lambda b: (0, 0)),
            pl.BlockSpec((_R, _R * U), lambda b: (0, 0)),
        ],
        out_specs=[
            pl.BlockSpec((_R, 1, N), lambda b: (b, 0, 0)),
            pl.BlockSpec((_R, 1, D), lambda b: (b, 0, 0)),
        ],
        out_shape=[
            jax.ShapeDtypeStruct((B, 1, N), jnp.float32),
            jax.ShapeDtypeStruct((B, 1, D), jnp.float32),
        ],
        compiler_params=pltpu.CompilerParams(
            dimension_semantics=("arbitrary",)),
    )(features, hidden3, wft, bf2, Wh, bh2, wblk)

    alpha = alpha2.reshape(B, N, 1)
    z = z3.reshape(B, D)
    return z, alpha


# speedup vs baseline: 1.0642x; 1.0642x over previous
import jax
import jax.numpy as jnp
from jax.experimental import pallas as pl
from jax.experimental.pallas import tpu as pltpu

_R = 4


def _hard_attention_kernel(feat_ref, hid_ref, wft_ref, bf_ref, wh_ref, bh_ref,
                           wblk_ref, alpha_ref, z_ref):
    b = pl.program_id(0)
    n = feat_ref.shape[1]
    dn = (((1,), (1,)), ((), ()))
    ones_row = jnp.ones((1, n), jnp.float32)

    us = []
    for r in range(_R):
        feat = feat_ref[r]
        hrow = hid_ref[pl.ds(b * _R + r, 1), 0, :]

        ft = jax.lax.dot_general(wft_ref[...], feat, dn,
                                 preferred_element_type=jnp.float32)
        hb_row = (jnp.dot(hrow, wh_ref[...],
                          preferred_element_type=jnp.float32)
                  + bh_ref[...] + bf_ref[...])
        hb_bc = jax.lax.dot_general(hb_row, ones_row, (((0,), (0,)), ((), ())),
                                    preferred_element_type=jnp.float32,
                                    precision=jax.lax.Precision.HIGHEST)
        us.append(jnp.tanh(ft + hb_bc))

    ucat = jnp.concatenate(us, axis=0)
    lo = jnp.dot(wblk_ref[...], ucat,
                 preferred_element_type=jnp.float32)

    m = jnp.max(lo, axis=1, keepdims=True)
    e = jnp.exp(lo - m)
    s = jnp.sum(e, axis=1, keepdims=True)
    alpha_ref[:, 0, :] = e * (1.0 / s)

    iota = jax.lax.broadcasted_iota(jnp.int32, (_R, n), 1)
    bidx = jnp.min(jnp.where(lo == m, iota, n), axis=1)
    for r in range(_R):
        z_ref[r] = feat_ref[r, pl.ds(bidx[r], 1), :]


def kernel(features, hidden, Wf, bf, Wh, bh, Ws, bs):
    B, N, D = features.shape
    H = hidden.shape[1]
    U = Wf.shape[1]

    hidden3 = hidden.reshape(B, 1, H)
    wft = Wf.T
    bf2 = bf.reshape(1, U)
    bh2 = bh.reshape(1, U)
    wblk = jnp.kron(jnp.eye(_R, dtype=jnp.float32), Ws.reshape(1, U))

    alpha2, z3 = pl.pallas_call(
        _hard_attention_kernel,
        grid=(B // _R,),
        in_specs=[
            pl.BlockSpec((_R, N, D), lambda b: (b, 0, 0)),
            pl.BlockSpec((B, 1, H), lambda b: (0, 0, 0)),
            pl.BlockSpec((U, D), lambda b: (0, 0)),
            pl.BlockSpec((1, U), lambda b: (0, 0)),
            pl.BlockSpec((H, U), lambda b: (0, 0)),
            pl.BlockSpec((1, U), lambda b: (0, 0)),
            pl.BlockSpec((_R, _R * U), lambda b: (0, 0)),
        ],
        out_specs=[
            pl.BlockSpec((_R, 1, N), lambda b: (b, 0, 0)),
            pl.BlockSpec((_R, 1, D), lambda b: (b, 0, 0)),
        ],
        out_shape=[
            jax.ShapeDtypeStruct((B, 1, N), jnp.float32),
            jax.ShapeDtypeStruct((B, 1, D), jnp.float32),
        ],
        compiler_params=pltpu.CompilerParams(
            dimension_semantics=("arbitrary",)),
    )(features, hidden3, wft, bf2, Wh, bh2, wblk)

    alpha = alpha2.reshape(B, N, 1)
    z = z3.reshape(B, D)
    return z, alpha
